# Initial kernel scaffold; baseline (speedup 1.0000x reference)
import jax
import jax.numpy as jnp
from jax.experimental import pallas as pl


def kernel(x, weight):
    raise NotImplementedError("write your pallas kernel here")


# speedup vs baseline: 1.8317x; 1.8317x over previous
import functools

import jax
import jax.numpy as jnp
from jax import lax
from jax.experimental import pallas as pl
from jax.experimental.pallas import tpu as pltpu
from jax.experimental.pallas import tpu_sc as plsc

EMBED_DIM = 64
NUM_WORKERS = 32
CHUNK = 512


def _gather_body(idx_hbm, table_hbm, out_hbm, idx_v, rows_v, gsem, *, b_per_w):
    n_chunks = b_per_w // CHUNK
    wid = lax.axis_index("s") * 2 + lax.axis_index("c")
    base = wid * b_per_w
    pltpu.sync_copy(idx_hbm.at[pl.ds(base, b_per_w)], idx_v)

    def body(j, carry):
        off = j * CHUNK
        pltpu.async_copy(
            table_hbm.at[idx_v.at[pl.ds(off, CHUNK)]], rows_v, gsem
        ).wait()
        pltpu.sync_copy(rows_v, out_hbm.at[pl.ds(base + off, CHUNK)])
        return carry

    lax.fori_loop(0, n_chunks, body, 0)


def kernel(x, weight):
    batch, hist = x.shape
    n = batch * hist
    b_per_w = n // NUM_WORKERS
    idx = x.reshape(n).astype(jnp.int32)

    mesh = plsc.VectorSubcoreMesh(core_axis_name="c", subcore_axis_name="s")
    gather = functools.partial(
        pl.kernel,
        mesh=mesh,
        out_type=jax.ShapeDtypeStruct((n, EMBED_DIM), jnp.float32),
        scratch_types=[
            pltpu.VMEM((b_per_w,), jnp.int32),
            pltpu.VMEM((CHUNK, EMBED_DIM), jnp.float32),
            pltpu.SemaphoreType.DMA,
        ],
        compiler_params=pltpu.CompilerParams(use_tc_tiling_on_sc=False),
    )(functools.partial(_gather_body, b_per_w=b_per_w))

    out = gather(idx, weight)
    return out.reshape(batch, hist, EMBED_DIM)


# speedup vs baseline: 1.8641x; 1.0177x over previous
import functools

import jax
import jax.numpy as jnp
from jax import lax
from jax.experimental import pallas as pl
from jax.experimental.pallas import tpu as pltpu
from jax.experimental.pallas import tpu_sc as plsc

EMBED_DIM = 64
NUM_WORKERS = 32
CHUNK = 512


def _gather_body(
    idx_hbm, table_hbm, out_hbm, idx_v, rows0, rows1, g0, g1, o0, o1, *, b_per_w
):
    n_chunks = b_per_w // CHUNK
    n_pairs = n_chunks // 2
    wid = lax.axis_index("s") * 2 + lax.axis_index("c")
    base = wid * b_per_w
    pltpu.sync_copy(idx_hbm.at[pl.ds(base, b_per_w)], idx_v)

    def gather(j, buf, sem):
        pltpu.async_copy(table_hbm.at[idx_v.at[pl.ds(j * CHUNK, CHUNK)]], buf, sem)

    def wait_gather(buf, sem):
        pltpu.make_async_copy(
            table_hbm.at[idx_v.at[pl.ds(0, CHUNK)]], buf, sem
        ).wait()

    def put(j, buf, sem):
        pltpu.async_copy(buf, out_hbm.at[pl.ds(base + j * CHUNK, CHUNK)], sem)

    def wait_put(buf, sem):
        pltpu.make_async_copy(buf, out_hbm.at[pl.ds(base, CHUNK)], sem).wait()

    gather(0, rows0, g0)
    gather(1, rows1, g1)

    def body(i, carry):
        j = i * 2
        wait_gather(rows0, g0)
        put(j, rows0, o0)
        wait_gather(rows1, g1)
        put(j + 1, rows1, o1)
        wait_put(rows0, o0)
        gather(j + 2, rows0, g0)
        wait_put(rows1, o1)
        gather(j + 3, rows1, g1)
        return carry

    lax.fori_loop(0, n_pairs - 1, body, 0)

    j = n_chunks - 2
    wait_gather(rows0, g0)
    put(j, rows0, o0)
    wait_gather(rows1, g1)
    put(j + 1, rows1, o1)
    wait_put(rows0, o0)
    wait_put(rows1, o1)


def kernel(x, weight):
    batch, hist = x.shape
    n = batch * hist
    b_per_w = n // NUM_WORKERS
    idx = x.reshape(n).astype(jnp.int32)

    mesh = plsc.VectorSubcoreMesh(core_axis_name="c", subcore_axis_name="s")
    gather = functools.partial(
        pl.kernel,
        mesh=mesh,
        out_type=jax.ShapeDtypeStruct((n, EMBED_DIM), jnp.float32),
        scratch_types=[
            pltpu.VMEM((b_per_w,), jnp.int32),
            pltpu.VMEM((CHUNK, EMBED_DIM), jnp.float32),
            pltpu.VMEM((CHUNK, EMBED_DIM), jnp.float32),
            pltpu.SemaphoreType.DMA,
            pltpu.SemaphoreType.DMA,
            pltpu.SemaphoreType.DMA,
            pltpu.SemaphoreType.DMA,
        ],
        compiler_params=pltpu.CompilerParams(use_tc_tiling_on_sc=False),
    )(functools.partial(_gather_body, b_per_w=b_per_w))

    out = gather(idx, weight)
    return out.reshape(batch, hist, EMBED_DIM)
